# Initial kernel scaffold; baseline (speedup 1.0000x reference)
import jax
import jax.numpy as jnp
from jax.experimental import pallas as pl


def kernel(inputs, adj, enc0_Wg, enc0_bg, enc0_Wc, enc0_bc, enc1_Wg, enc1_bg, enc1_Wc, enc1_bc, dec0_Wg, dec0_bg, dec0_Wc, dec0_bc, dec1_Wg, dec1_bg, dec1_Wc, dec1_bc, proj_W, proj_b):
    raise NotImplementedError("write your pallas kernel here")


# speedup vs baseline: 2.9446x; 2.9446x over previous
import jax
import jax.numpy as jnp
from jax.experimental import pallas as pl
from jax.experimental.pallas import tpu as pltpu

N = 512
B = 16
INP = 2
OUP = 1
UNITS = 64
SEQ = 12
HOR = 12
K = 2
M = K + 1


def _reorder_W(W):
    C = W.shape[0] // M
    return W.reshape(C, M, -1).transpose(1, 0, 2)


def _split_W(W, Cx):
    Wr = _reorder_W(W)
    return Wr[:, :Cx, :], Wr[:, Cx:, :]


def _step_kernel(adjT_ref, xseq_ref,
                 Wgx_ref, Wgh_ref, bg0_ref, Wcx_ref, Wch_ref, bc0_ref,
                 Wg1_ref, bg1_ref, Wcx1_ref, Wch1_ref, bc1_ref,
                 pW_ref, pb_ref, out_ref,
                 h0_ref, h1_ref, din_ref, sup_ref):
    t = pl.program_id(0)

    @pl.when(t == 0)
    def _init():
        adjT = adjT_ref[:]
        d = jnp.sum(adjT, axis=0, keepdims=True)
        dinv = jnp.where(d > 0, 1.0 / d, 0.0)
        sup_ref[:] = adjT * dinv
        h0_ref[:] = jnp.zeros((B, N, UNITS), jnp.float32)
        h1_ref[:] = jnp.zeros((B, N, UNITS), jnp.float32)
        din_ref[:] = jnp.zeros((B, N), jnp.float32)

    def smat(x):
        sup = sup_ref[:]
        return jnp.stack(
            [jnp.dot(sup, x[b], preferred_element_type=jnp.float32)
             for b in range(B)], axis=0)

    def wmm(x, W_ref, m, C):
        Wm = W_ref[0, m]
        y = jnp.dot(x.reshape(B * N, C), Wm,
                    preferred_element_type=jnp.float32)
        return y.reshape(B, N, Wm.shape[-1])

    def zsmall(xc, W_ref, m):
        acc = xc[0][:, :, None] * W_ref[0, m, 0:1, :].reshape(1, 1, -1)
        for c in range(1, INP):
            acc = acc + xc[c][:, :, None] * W_ref[0, m, c:c + 1, :].reshape(1, 1, -1)
        return acc

    def xpart_horner(xc, Wa_ref, Wb_ref, outa):
        z0 = jnp.concatenate([zsmall(xc, Wa_ref, 0), zsmall(xc, Wb_ref, 0)],
                             axis=-1)
        z1 = jnp.concatenate([zsmall(xc, Wa_ref, 1), zsmall(xc, Wb_ref, 1)],
                             axis=-1)
        z2 = jnp.concatenate([zsmall(xc, Wa_ref, 2), zsmall(xc, Wb_ref, 2)],
                             axis=-1)
        acc = z0 - z2 + smat(z1 + 2.0 * smat(z2))
        return acc[..., :outa], acc[..., outa:]

    def cheb1(x, Wa_ref):
        aa = wmm(x, Wa_ref, 0, UNITS)
        x1 = smat(x)
        aa = aa + wmm(x1, Wa_ref, 1, UNITS)
        x2 = 2.0 * smat(x1) - x
        return aa + wmm(x2, Wa_ref, 2, UNITS)

    enc_phase = t < SEQ
    xc0 = jnp.where(enc_phase, xseq_ref[0, 0], din_ref[:])
    xc1 = jnp.where(enc_phase, xseq_ref[1, 0], 0.0)
    xc = [xc0, xc1]

    st = h0_ref[:]
    gx, cx = xpart_horner(xc, Wgx_ref, Wcx_ref, 2 * UNITS)
    gh = cheb1(st, Wgh_ref)
    g = jax.nn.sigmoid(gx + gh + bg0_ref[0, 0].reshape(1, 1, 2 * UNITS))
    r = g[..., :UNITS]
    u = g[..., UNITS:]
    ch = cheb1(r * st, Wch_ref)
    c = jnp.tanh(cx + ch + bc0_ref[0, 0].reshape(1, 1, UNITS))
    h0 = u * st + (1.0 - u) * c
    h0_ref[:] = h0

    st1 = h1_ref[:]
    xs = jnp.concatenate([h0, st1], axis=-1)
    t1 = smat(xs)
    t2 = 2.0 * smat(t1) - xs
    g1 = wmm(xs, Wg1_ref, 0, 2 * UNITS) \
        + wmm(t1, Wg1_ref, 1, 2 * UNITS) \
        + wmm(t2, Wg1_ref, 2, 2 * UNITS)
    g1 = jax.nn.sigmoid(g1 + bg1_ref[0, 0].reshape(1, 1, 2 * UNITS))
    r1 = g1[..., :UNITS]
    u1 = g1[..., UNITS:]
    cx1 = wmm(h0, Wcx1_ref, 0, UNITS) \
        + wmm(t1[:, :, :UNITS], Wcx1_ref, 1, UNITS) \
        + wmm(t2[:, :, :UNITS], Wcx1_ref, 2, UNITS)
    ch1 = cheb1(r1 * st1, Wch1_ref)
    c1 = jnp.tanh(cx1 + ch1 + bc1_ref[0, 0].reshape(1, 1, UNITS))
    h1 = u1 * st1 + (1.0 - u1) * c1
    h1_ref[:] = h1

    proj = jnp.sum(h1 * pW_ref[:].reshape(1, 1, UNITS), axis=-1) \
        + pb_ref[0, 0]
    out_ref[0] = proj

    @pl.when(jnp.logical_not(enc_phase))
    def _feedback():
        din_ref[:] = proj


@jax.jit
def kernel(inputs, adj, enc0_Wg, enc0_bg, enc0_Wc, enc0_bc, enc1_Wg, enc1_bg,
           enc1_Wc, enc1_bc, dec0_Wg, dec0_bg, dec0_Wc, dec0_bc, dec1_Wg,
           dec1_bg, dec1_Wc, dec1_bc, proj_W, proj_b):
    xseq = inputs.reshape(B, N, INP, SEQ).transpose(2, 3, 0, 1)
    e0Wgx, e0Wgh = _split_W(enc0_Wg, INP)
    e0Wcx, e0Wch = _split_W(enc0_Wc, INP)
    d0Wgx, d0Wgh = _split_W(dec0_Wg, OUP)
    d0Wcx, d0Wch = _split_W(dec0_Wc, OUP)
    pad = lambda w: jnp.concatenate(
        [w, jnp.zeros((M, INP - OUP, w.shape[-1]), w.dtype)], axis=1)
    Wgx = jnp.stack([e0Wgx, pad(d0Wgx)])
    Wcx = jnp.stack([e0Wcx, pad(d0Wcx)])
    Wgh = jnp.stack([e0Wgh, d0Wgh])
    Wch = jnp.stack([e0Wch, d0Wch])
    bg0 = jnp.stack([enc0_bg, dec0_bg])
    bc0 = jnp.stack([enc0_bc, dec0_bc])
    Wg1 = jnp.stack([_reorder_W(enc1_Wg), _reorder_W(dec1_Wg)])
    e1Wcx, e1Wch = _split_W(enc1_Wc, UNITS)
    d1Wcx, d1Wch = _split_W(dec1_Wc, UNITS)
    Wcx1 = jnp.stack([e1Wcx, d1Wcx])
    Wch1 = jnp.stack([e1Wch, d1Wch])
    bg1 = jnp.stack([enc1_bg, dec1_bg])
    bc1 = jnp.stack([enc1_bc, dec1_bc])

    full = lambda shape: pl.BlockSpec(shape, lambda t: (0,) * len(shape))
    phase = lambda shape: pl.BlockSpec(
        (1,) + shape, lambda t: (t // SEQ,) + (0,) * len(shape))
    b3 = lambda b: b.reshape(2, 1, -1)

    out = pl.pallas_call(
        _step_kernel,
        grid=(SEQ + HOR,),
        in_specs=[
            full((N, N)),
            pl.BlockSpec((INP, 1, B, N),
                         lambda t: (0, jnp.minimum(t, SEQ - 1), 0, 0)),
            phase((M, INP, 2 * UNITS)),
            phase((M, UNITS, 2 * UNITS)),
            phase((1, 2 * UNITS)),
            phase((M, INP, UNITS)),
            phase((M, UNITS, UNITS)),
            phase((1, UNITS)),
            phase((M, 2 * UNITS, 2 * UNITS)),
            phase((1, 2 * UNITS)),
            phase((M, UNITS, UNITS)),
            phase((M, UNITS, UNITS)),
            phase((1, UNITS)),
            full((1, UNITS)),
            full((1, 1)),
        ],
        out_specs=pl.BlockSpec((1, B, N),
                               lambda t: (jnp.maximum(t - SEQ, 0), 0, 0)),
        out_shape=jax.ShapeDtypeStruct((HOR, B, N), jnp.float32),
        scratch_shapes=[pltpu.VMEM((B, N, UNITS), jnp.float32),
                        pltpu.VMEM((B, N, UNITS), jnp.float32),
                        pltpu.VMEM((B, N), jnp.float32),
                        pltpu.VMEM((N, N), jnp.float32)],
        compiler_params=pltpu.CompilerParams(
            vmem_limit_bytes=100 * 1024 * 1024,
            dimension_semantics=("arbitrary",)),
    )(adj.T, xseq, Wgx, Wgh, b3(bg0), Wcx, Wch, b3(bc0),
      Wg1, b3(bg1), Wcx1, Wch1, b3(bc1), proj_W.reshape(1, UNITS),
      proj_b.reshape(1, 1))
    return out.reshape(HOR, B, N * OUP)


# speedup vs baseline: 2.9635x; 1.0064x over previous
import jax
import jax.numpy as jnp
from jax.experimental import pallas as pl
from jax.experimental.pallas import tpu as pltpu

N = 512
B = 16
INP = 2
OUP = 1
UNITS = 64
SEQ = 12
HOR = 12
K = 2
M = K + 1


def _reorder_W(W):
    C = W.shape[0] // M
    return W.reshape(C, M, -1).transpose(1, 0, 2)


def _split_W(W, Cx):
    Wr = _reorder_W(W)
    return Wr[:, :Cx, :], Wr[:, Cx:, :]


def _step_kernel(adjT_ref, xseq_ref,
                 Wgx_ref, Wgh_ref, bg0_ref, Wcx_ref, Wch_ref, bc0_ref,
                 Wg1_ref, bg1_ref, Wcx1_ref, Wch1_ref, bc1_ref,
                 pW_ref, pb_ref, out_ref,
                 h0_ref, h1_ref, din_ref, sup_ref):
    t = pl.program_id(0)

    @pl.when(t == 0)
    def _init():
        adjT = adjT_ref[:]
        d = jnp.sum(adjT, axis=0, keepdims=True)
        dinv = jnp.where(d > 0, 1.0 / d, 0.0)
        sup_ref[:] = (adjT * dinv).astype(jnp.bfloat16)
        h0_ref[:] = jnp.zeros((B, N, UNITS), jnp.float32)
        h1_ref[:] = jnp.zeros((B, N, UNITS), jnp.float32)
        din_ref[:] = jnp.zeros((B, N), jnp.float32)

    def smat(x):
        sup = sup_ref[:]
        xb = x.astype(jnp.bfloat16)
        return jnp.stack(
            [jnp.dot(sup, xb[b], preferred_element_type=jnp.float32)
             for b in range(B)], axis=0)

    def wmm(x, W_ref, m, C):
        Wm = W_ref[0, m]
        y = jnp.dot(x.reshape(B * N, C), Wm,
                    preferred_element_type=jnp.float32)
        return y.reshape(B, N, Wm.shape[-1])

    def zsmall(xc, W_ref, m):
        acc = xc[0][:, :, None] * W_ref[0, m, 0:1, :].reshape(1, 1, -1)
        for c in range(1, INP):
            acc = acc + xc[c][:, :, None] * W_ref[0, m, c:c + 1, :].reshape(1, 1, -1)
        return acc

    def xpart_horner(xc, Wa_ref, Wb_ref, outa):
        z0 = jnp.concatenate([zsmall(xc, Wa_ref, 0), zsmall(xc, Wb_ref, 0)],
                             axis=-1)
        z1 = jnp.concatenate([zsmall(xc, Wa_ref, 1), zsmall(xc, Wb_ref, 1)],
                             axis=-1)
        z2 = jnp.concatenate([zsmall(xc, Wa_ref, 2), zsmall(xc, Wb_ref, 2)],
                             axis=-1)
        acc = z0 - z2 + smat(z1 + 2.0 * smat(z2))
        return acc[..., :outa], acc[..., outa:]

    def cheb1(x, Wa_ref):
        aa = wmm(x, Wa_ref, 0, UNITS)
        x1 = smat(x)
        aa = aa + wmm(x1, Wa_ref, 1, UNITS)
        x2 = 2.0 * smat(x1) - x
        return aa + wmm(x2, Wa_ref, 2, UNITS)

    enc_phase = t < SEQ
    xc0 = jnp.where(enc_phase, xseq_ref[0, 0], din_ref[:])
    xc1 = jnp.where(enc_phase, xseq_ref[1, 0], 0.0)
    xc = [xc0, xc1]

    st = h0_ref[:]
    gx, cx = xpart_horner(xc, Wgx_ref, Wcx_ref, 2 * UNITS)
    gh = cheb1(st, Wgh_ref)
    g = jax.nn.sigmoid(gx + gh + bg0_ref[0, 0].reshape(1, 1, 2 * UNITS))
    r = g[..., :UNITS]
    u = g[..., UNITS:]
    ch = cheb1(r * st, Wch_ref)
    c = jnp.tanh(cx + ch + bc0_ref[0, 0].reshape(1, 1, UNITS))
    h0 = u * st + (1.0 - u) * c
    h0_ref[:] = h0

    st1 = h1_ref[:]
    xs = jnp.concatenate([h0, st1], axis=-1)
    t1 = smat(xs)
    t2 = 2.0 * smat(t1) - xs
    g1 = wmm(xs, Wg1_ref, 0, 2 * UNITS) \
        + wmm(t1, Wg1_ref, 1, 2 * UNITS) \
        + wmm(t2, Wg1_ref, 2, 2 * UNITS)
    g1 = jax.nn.sigmoid(g1 + bg1_ref[0, 0].reshape(1, 1, 2 * UNITS))
    r1 = g1[..., :UNITS]
    u1 = g1[..., UNITS:]
    cx1 = wmm(h0, Wcx1_ref, 0, UNITS) \
        + wmm(t1[:, :, :UNITS], Wcx1_ref, 1, UNITS) \
        + wmm(t2[:, :, :UNITS], Wcx1_ref, 2, UNITS)
    ch1 = cheb1(r1 * st1, Wch1_ref)
    c1 = jnp.tanh(cx1 + ch1 + bc1_ref[0, 0].reshape(1, 1, UNITS))
    h1 = u1 * st1 + (1.0 - u1) * c1
    h1_ref[:] = h1

    proj = jnp.sum(h1 * pW_ref[:].reshape(1, 1, UNITS), axis=-1) \
        + pb_ref[0, 0]
    out_ref[0] = proj

    @pl.when(jnp.logical_not(enc_phase))
    def _feedback():
        din_ref[:] = proj


@jax.jit
def kernel(inputs, adj, enc0_Wg, enc0_bg, enc0_Wc, enc0_bc, enc1_Wg, enc1_bg,
           enc1_Wc, enc1_bc, dec0_Wg, dec0_bg, dec0_Wc, dec0_bc, dec1_Wg,
           dec1_bg, dec1_Wc, dec1_bc, proj_W, proj_b):
    xseq = inputs.reshape(B, N, INP, SEQ).transpose(2, 3, 0, 1)
    e0Wgx, e0Wgh = _split_W(enc0_Wg, INP)
    e0Wcx, e0Wch = _split_W(enc0_Wc, INP)
    d0Wgx, d0Wgh = _split_W(dec0_Wg, OUP)
    d0Wcx, d0Wch = _split_W(dec0_Wc, OUP)
    pad = lambda w: jnp.concatenate(
        [w, jnp.zeros((M, INP - OUP, w.shape[-1]), w.dtype)], axis=1)
    Wgx = jnp.stack([e0Wgx, pad(d0Wgx)])
    Wcx = jnp.stack([e0Wcx, pad(d0Wcx)])
    Wgh = jnp.stack([e0Wgh, d0Wgh])
    Wch = jnp.stack([e0Wch, d0Wch])
    bg0 = jnp.stack([enc0_bg, dec0_bg])
    bc0 = jnp.stack([enc0_bc, dec0_bc])
    Wg1 = jnp.stack([_reorder_W(enc1_Wg), _reorder_W(dec1_Wg)])
    e1Wcx, e1Wch = _split_W(enc1_Wc, UNITS)
    d1Wcx, d1Wch = _split_W(dec1_Wc, UNITS)
    Wcx1 = jnp.stack([e1Wcx, d1Wcx])
    Wch1 = jnp.stack([e1Wch, d1Wch])
    bg1 = jnp.stack([enc1_bg, dec1_bg])
    bc1 = jnp.stack([enc1_bc, dec1_bc])

    full = lambda shape: pl.BlockSpec(shape, lambda t: (0,) * len(shape))
    phase = lambda shape: pl.BlockSpec(
        (1,) + shape, lambda t: (t // SEQ,) + (0,) * len(shape))
    b3 = lambda b: b.reshape(2, 1, -1)

    out = pl.pallas_call(
        _step_kernel,
        grid=(SEQ + HOR,),
        in_specs=[
            full((N, N)),
            pl.BlockSpec((INP, 1, B, N),
                         lambda t: (0, jnp.minimum(t, SEQ - 1), 0, 0)),
            phase((M, INP, 2 * UNITS)),
            phase((M, UNITS, 2 * UNITS)),
            phase((1, 2 * UNITS)),
            phase((M, INP, UNITS)),
            phase((M, UNITS, UNITS)),
            phase((1, UNITS)),
            phase((M, 2 * UNITS, 2 * UNITS)),
            phase((1, 2 * UNITS)),
            phase((M, UNITS, UNITS)),
            phase((M, UNITS, UNITS)),
            phase((1, UNITS)),
            full((1, UNITS)),
            full((1, 1)),
        ],
        out_specs=pl.BlockSpec((1, B, N),
                               lambda t: (jnp.maximum(t - SEQ, 0), 0, 0)),
        out_shape=jax.ShapeDtypeStruct((HOR, B, N), jnp.float32),
        scratch_shapes=[pltpu.VMEM((B, N, UNITS), jnp.float32),
                        pltpu.VMEM((B, N, UNITS), jnp.float32),
                        pltpu.VMEM((B, N), jnp.float32),
                        pltpu.VMEM((N, N), jnp.bfloat16)],
        compiler_params=pltpu.CompilerParams(
            vmem_limit_bytes=100 * 1024 * 1024,
            dimension_semantics=("arbitrary",)),
    )(adj.T, xseq, Wgx, Wgh, b3(bg0), Wcx, Wch, b3(bc0),
      Wg1, b3(bg1), Wcx1, Wch1, b3(bc1), proj_W.reshape(1, UNITS),
      proj_b.reshape(1, 1))
    return out.reshape(HOR, B, N * OUP)
